# baseline (speedup 1.0000x reference)
import functools

import numpy as np
import jax
import jax.numpy as jnp
from jax import lax
from jax.experimental import pallas as pl
from jax.experimental.pallas import tpu as pltpu
from jax.experimental.pallas import tpu_sc as plsc

N = 10000
E = 160000
MUL = 16
D = 64
W128 = 128
CUTOFF = 1.0
EPS = 1e-8

NC = 2
NS = 16
NW = NC * NS
CHUNK = 128
CH_PER_GROUP = 5
GROUPS = 8
CH_W = CH_PER_GROUP * GROUPS
E_W = CH_W * CHUNK
E_PAD = NW * E_W
GROUP_E = CH_PER_GROUP * CHUNK
TOT_CH = NW * CH_W
N_PAD = 10240
TRASH = N_PAD - 1
STRIPE = N_PAD // NS

EB = 2000
N_EB = E // EB

_SINP = (3.1415925994720157, -5.167708081845069, 2.550050988760038,
         -0.5981614666896184, 0.07744687538920765)

_C3 = float(1.0 / np.sqrt(3.0))
_P0 = float(1.0 / np.sqrt(2.0 * MUL))
_P1C = float(np.sqrt(3.0 / (2.0 * MUL)) / np.sqrt(3.0))
_SCL = float(1.0 / np.sqrt(MUL))

_P_np = np.zeros((64, 64), np.float32)
for _j in range(16):
    _P_np[_j, _j] = 1.0
for _u in range(16):
    for _k in range(3):
        _P_np[16 + 3 * _u + _k, 16 + 16 * _k + _u] = 1.0

_R4_np = np.zeros((64, 1024), np.float32)
for _g in range(4):
    for _u in range(16):
        _R4_np[_g * 16 + _u, _g * 256 + _u * 16:_g * 256 + (_u + 1) * 16] = 1.0
_S4_np = np.zeros((1024, 64), np.float32)
for _q, _sc in enumerate([_P0, _P1C, _P1C, _P1C]):
    for _u in range(16):
        for _w in range(16):
            _S4_np[_q * 256 + _u * 16 + _w, _q * 16 + _w] = _sc


def _silu(v):
    return v * jax.nn.sigmoid(v)


def _prep_body(x_ref, w_ref, b_ref, p_ref, out_ref):
    f32 = jnp.float32
    P = p_ref[...]
    xp = jnp.dot(x_ref[...], P, preferred_element_type=f32)
    w_p = jnp.dot(w_ref[...], P, preferred_element_type=f32)
    b_p = jnp.dot(b_ref[...], P, preferred_element_type=f32)
    s = xp[:, :MUL]
    mean = jnp.mean(s, axis=1, keepdims=True)
    var = jnp.mean((s - mean) ** 2, axis=1, keepdims=True)
    sn = (s - mean) * lax.rsqrt(var + EPS)
    vx = xp[:, 16:32]
    vy = xp[:, 32:48]
    vz = xp[:, 48:64]
    ninv = lax.rsqrt((vx * vx + vy * vy + vz * vz) * (1.0 / 3.0) + EPS)
    out = jnp.concatenate([sn, vx * ninv, vy * ninv, vz * ninv], axis=1)
    out = out * w_p + b_p
    out_ref[...] = jnp.concatenate(
        [out, jnp.zeros((N, 64), f32)], axis=1)


def _prep(x, nw, nb, Pm):
    return pl.pallas_call(
        _prep_body,
        out_shape=jax.ShapeDtypeStruct((N, W128), jnp.float32),
    )(x, nw, nb, Pm)


def _gather_sc(table, idx2d):
    mesh = plsc.VectorSubcoreMesh(core_axis_name="c", subcore_axis_name="s")

    @functools.partial(
        pl.kernel,
        mesh=mesh,
        out_type=jax.ShapeDtypeStruct((E_PAD, W128), jnp.float32),
        scratch_types=[
            pltpu.VMEM((CH_W, CHUNK), jnp.int32),
            pltpu.VMEM((GROUP_E, W128), jnp.float32),
            pltpu.SemaphoreType.DMA,
        ],
    )
    def gk(tab_hbm, idx_hbm, out_hbm, idx_v, gbuf, sem):
        c = lax.axis_index("c")
        s = lax.axis_index("s")
        wid = s * NC + c
        pltpu.sync_copy(idx_hbm.at[pl.ds(wid * CH_W, CH_W), :], idx_v)
        ebase = wid * E_W
        for g in range(GROUPS):
            cps = []
            for j in range(CH_PER_GROUP):
                cps.append(pltpu.async_copy(
                    tab_hbm.at[idx_v.at[g * CH_PER_GROUP + j]],
                    gbuf.at[pl.ds(j * CHUNK, CHUNK), :],
                    sem,
                ))
            for cp in cps:
                cp.wait()
            pltpu.sync_copy(
                gbuf, out_hbm.at[pl.ds(ebase + g * GROUP_E, GROUP_E), :])

    return gk(table, idx2d)


def _main_body(rbf_ref, aux_ref, xs_ref, w1_ref, b1_ref, w2_ref,
               b2_ref, w3_ref, b3_ref, gw1_ref, gb1_ref, gw2_ref, gb2_ref,
               r4_ref, s4_ref, out_ref):
    f32 = jnp.float32
    bf16 = jnp.bfloat16
    rbf = rbf_ref[...]
    h = _silu(jnp.dot(rbf, w1_ref[...], preferred_element_type=f32)
              + b1_ref[...])
    h = _silu(jnp.dot(h, w2_ref[...], preferred_element_type=f32)
              + b2_ref[...])
    tpw = jnp.dot(h.astype(bf16), w3_ref[...],
                  preferred_element_type=f32) + b3_ref[...]

    xs = xs_ref[...]
    aux = aux_ref[...]
    sh0 = aux[:, 0:1]
    shx = aux[:, 1:2]
    shy = aux[:, 2:3]
    shz = aux[:, 3:4]
    elen = aux[:, 4:5]
    t = elen - 0.5
    t2 = t * t
    sp = t * (_SINP[0] + t2 * (_SINP[1] + t2 * (_SINP[2] + t2 * (
        _SINP[3] + t2 * _SINP[4]))))
    cut = (0.5 - 0.5 * sp) * (elen <= CUTOFF).astype(f32)

    XR = jnp.dot(xs[:, :64].astype(bf16), r4_ref[...],
                 preferred_element_type=f32)
    sr = XR[:, 0:256]
    vxr = XR[:, 256:512]
    vyr = XR[:, 512:768]
    vzr = XR[:, 768:1024]

    ar = sr * sh0
    br = vxr * shx + vyr * shy + vzr * shz
    t0 = tpw[:, 0:256] * ar + _C3 * (tpw[:, 768:1024] * br)
    c011 = tpw[:, 256:512] * sr
    tq = tpw[:, 512:768]
    txp = c011 * shx + (tq * vxr) * sh0
    typ = c011 * shy + (tq * vyr) * sh0
    tzp = c011 * shz + (tq * vzr) * sh0
    T = jnp.concatenate([t0, txp, typ, tzp], axis=1)
    m = jnp.dot(T, s4_ref[...], preferred_element_type=f32)

    g = _silu(jnp.dot(rbf, gw1_ref[...], preferred_element_type=f32)
              + gb1_ref[...])
    gw = jax.nn.sigmoid(jnp.dot(g, gw2_ref[...], preferred_element_type=f32)
                        + gb2_ref[...])
    ew = cut * gw

    m = m * ew
    ew16 = jnp.broadcast_to(ew, (m.shape[0], 16))
    pad48 = jnp.zeros((m.shape[0], 48), f32)
    out_ref[...] = jnp.concatenate([m, ew16, pad48], axis=1)


def _main(rbf, aux, xs, w1, b1, w2, b2, w3, b3, gw1, gb1, gw2, gb2,
          R4m, S4m):
    blk = lambda shp: pl.BlockSpec(shp, lambda i: (0, 0))
    ebk = lambda w: pl.BlockSpec((EB, w), lambda i: (i, 0))
    return pl.pallas_call(
        _main_body,
        grid=(N_EB,),
        in_specs=[
            ebk(16), ebk(8), ebk(W128),
            blk((16, 64)), blk((1, 64)), blk((64, 64)), blk((1, 64)),
            blk((64, 1024)), blk((1, 1024)),
            blk((16, 64)), blk((1, 64)), blk((64, 1)), blk((1, 1)),
            blk((64, 1024)), blk((1024, 64)),
        ],
        out_specs=pl.BlockSpec((EB, W128), lambda i: (i, 0)),
        out_shape=jax.ShapeDtypeStruct((E_PAD, W128), jnp.float32),
    )(rbf, aux, xs, w1, b1, w2, b2, w3, b3, gw1, gb1, gw2, gb2,
      R4m, S4m)


def _scatter_sc(m_ext, dst2d, zstripe):
    mesh = plsc.VectorSubcoreMesh(core_axis_name="c", subcore_axis_name="s")

    @functools.partial(
        pl.kernel,
        mesh=mesh,
        out_type=jax.ShapeDtypeStruct((NC, N_PAD, W128), jnp.float32),
        scratch_types=[
            pltpu.VMEM((CH_W, CHUNK), jnp.int32),
            pltpu.VMEM((CHUNK, W128), jnp.float32),
            pltpu.VMEM((CHUNK, W128), jnp.float32),
            pltpu.VMEM_SHARED((N_PAD, W128), jnp.float32),
            pltpu.SemaphoreType.DMA,
            pltpu.SemaphoreType.DMA,
            pltpu.SemaphoreType.DMA,
            pltpu.SemaphoreType.DMA,
        ],
    )
    def sk(m_hbm, idx_hbm, z_hbm, out_hbm, idx_v, mb0, mb1, acc,
           ls0, ls1, as0, as1):
        c = lax.axis_index("c")
        s = lax.axis_index("s")
        wid = s * NC + c
        pltpu.sync_copy(idx_hbm.at[pl.ds(wid * CH_W, CH_W), :], idx_v)
        pltpu.sync_copy(z_hbm, acc.at[pl.ds(s * STRIPE, STRIPE), :])
        plsc.subcore_barrier()
        ebase = wid * E_W
        bufs = (mb0, mb1)
        lsems = (ls0, ls1)
        asems = (as0, as1)
        lds = [None, None]
        ads = [None, None]

        def load(t):
            q = t % 2
            lds[q] = pltpu.async_copy(
                m_hbm.at[pl.ds(ebase + t * CHUNK, CHUNK), :], bufs[q],
                lsems[q])

        load(0)
        for j in range(CH_W):
            p = j % 2
            nj = j + 1
            if nj < CH_W:
                q = nj % 2
                if ads[q] is not None:
                    ads[q].wait()
                    ads[q] = None
                load(nj)
            lds[p].wait()
            ads[p] = pltpu.async_copy(
                bufs[p], acc.at[idx_v.at[j]], asems[p], add=True)
        for q in range(2):
            if ads[q] is not None:
                ads[q].wait()
        plsc.subcore_barrier()
        pltpu.sync_copy(
            acc.at[pl.ds(s * STRIPE, STRIPE), :],
            out_hbm.at[c, pl.ds(s * STRIPE, STRIPE), :],
        )

    return sk(m_ext, dst2d, zstripe)


def _epi_body(aggc_ref, x_ref, xn_ref, mws_ref, mwg_ref, mwv_ref,
              uw0_ref, uw1_ref, sw0_ref, sw1_ref, rs_ref, pt_ref, out_ref):
    f32 = jnp.float32
    agg = aggc_ref[0] + aggc_ref[1]
    agg = agg[:N, :]
    den = jnp.maximum(agg[:, 64:65], 1e-8)
    a = agg[:, :64] / den
    a_s = a[:, :16]
    a_vx = a[:, 16:32]
    a_vy = a[:, 32:48]
    a_vz = a[:, 48:64]

    scal = _silu(jnp.dot(a_s, mws_ref[...], preferred_element_type=f32)
                 * _SCL)
    gts = jax.nn.sigmoid(jnp.dot(a_s, mwg_ref[...],
                                 preferred_element_type=f32) * _SCL)
    mwv = mwv_ref[...]
    vex = gts * (jnp.dot(a_vx, mwv, preferred_element_type=f32) * _SCL)
    vey = gts * (jnp.dot(a_vy, mwv, preferred_element_type=f32) * _SCL)
    vez = gts * (jnp.dot(a_vz, mwv, preferred_element_type=f32) * _SCL)

    xn = xn_ref[...]
    sw0 = sw0_ref[...]
    sw1 = sw1_ref[...]
    uw0 = uw0_ref[...]
    uw1 = uw1_ref[...]
    o_s = (jnp.dot(xn[:, :16], sw0, preferred_element_type=f32)
           + jnp.dot(scal, uw0, preferred_element_type=f32)) * _SCL
    o_vx = (jnp.dot(xn[:, 16:32], sw1, preferred_element_type=f32)
            + jnp.dot(vex, uw1, preferred_element_type=f32)) * _SCL
    o_vy = (jnp.dot(xn[:, 32:48], sw1, preferred_element_type=f32)
            + jnp.dot(vey, uw1, preferred_element_type=f32)) * _SCL
    o_vz = (jnp.dot(xn[:, 48:64], sw1, preferred_element_type=f32)
            + jnp.dot(vez, uw1, preferred_element_type=f32)) * _SCL
    out_p = jnp.concatenate([o_s, o_vx, o_vy, o_vz], axis=1)
    out = jnp.dot(out_p, pt_ref[...], preferred_element_type=f32)
    out_ref[...] = x_ref[...] + rs_ref[0, 0] * out


def _epilogue(aggc, x, xn_p, mws, mwg, mwv, uw0, uw1, sw0, sw1, rs, PTm):
    return pl.pallas_call(
        _epi_body,
        out_shape=jax.ShapeDtypeStruct((N, D), jnp.float32),
    )(aggc, x, xn_p, mws, mwg, mwv, uw0, uw1, sw0, sw1, rs, PTm)


def kernel(x, edge_src, edge_dst, edge_sh, edge_rbf, edge_len, norm_w,
           norm_b, mlp_w1, mlp_b1, mlp_w2, mlp_b2, mlp_w3, mlp_b3, gate_w1,
           gate_b1, gate_w2, gate_b2, msg_ws, msg_wg, msg_wv, upd_w0, upd_w1,
           self_w0, self_w1, res_scale):
    f32 = jnp.float32
    pad = E_PAD - E
    src_p = jnp.pad(edge_src.astype(jnp.int32), (0, pad)).reshape(TOT_CH,
                                                                  CHUNK)
    dst_p = jnp.pad(edge_dst.astype(jnp.int32), (0, pad),
                    constant_values=TRASH).reshape(TOT_CH, CHUNK)

    zstripe = jnp.zeros((STRIPE, W128), f32)
    Pm = jnp.asarray(_P_np)
    PTm = jnp.asarray(_P_np.T)
    R4m = jnp.asarray(_R4_np, dtype=jnp.bfloat16)
    S4m = jnp.asarray(_S4_np)

    xn_p = _prep(x, norm_w.reshape(1, D), norm_b.reshape(1, D), Pm)
    aux = jnp.concatenate(
        [edge_sh, edge_len[:, None], jnp.zeros((E, 3), f32)], axis=1)
    xs = _gather_sc(xn_p, src_p)
    m_ext = _main(edge_rbf, aux, xs,
                  mlp_w1, mlp_b1.reshape(1, -1), mlp_w2,
                  mlp_b2.reshape(1, -1), mlp_w3.astype(jnp.bfloat16),
                  mlp_b3.reshape(1, -1),
                  gate_w1, gate_b1.reshape(1, -1), gate_w2,
                  gate_b2.reshape(1, -1), R4m, S4m)
    aggc = _scatter_sc(m_ext, dst_p, zstripe)
    return _epilogue(aggc, x, xn_p, msg_ws, msg_wg, msg_wv, upd_w0,
                     upd_w1, self_w0, self_w1,
                     res_scale.reshape(1, 1), PTm)


# speedup vs baseline: 1.0468x; 1.0468x over previous
import functools

import numpy as np
import jax
import jax.numpy as jnp
from jax import lax
from jax.experimental import pallas as pl
from jax.experimental.pallas import tpu as pltpu
from jax.experimental.pallas import tpu_sc as plsc

N = 10000
E = 160000
MUL = 16
D = 64
W128 = 128
CUTOFF = 1.0
EPS = 1e-8

NC = 2
NS = 16
NW = NC * NS
CHUNK = 128
CH_PER_GROUP = 5
GROUPS = 8
CH_W = CH_PER_GROUP * GROUPS
E_W = CH_W * CHUNK
E_PAD = NW * E_W
GROUP_E = CH_PER_GROUP * CHUNK
TOT_CH = NW * CH_W
N_PAD = 10240
TRASH = N_PAD - 1
STRIPE = N_PAD // NS

EB = 2000
N_EB = E // EB

_SINP = (3.1415925994720157, -5.167708081845069, 2.550050988760038,
         -0.5981614666896184, 0.07744687538920765)

_C3 = float(1.0 / np.sqrt(3.0))
_P0 = float(1.0 / np.sqrt(2.0 * MUL))
_P1C = float(np.sqrt(3.0 / (2.0 * MUL)) / np.sqrt(3.0))
_SCL = float(1.0 / np.sqrt(MUL))

_P_np = np.zeros((64, 64), np.float32)
for _j in range(16):
    _P_np[_j, _j] = 1.0
for _u in range(16):
    for _k in range(3):
        _P_np[16 + 3 * _u + _k, 16 + 16 * _k + _u] = 1.0

_R4_np = np.zeros((64, 1024), np.float32)
for _g in range(4):
    for _u in range(16):
        _R4_np[_g * 16 + _u, _g * 256 + _u * 16:_g * 256 + (_u + 1) * 16] = 1.0
_S4_np = np.zeros((1024, 64), np.float32)
for _q, _sc in enumerate([_P0, _P1C, _P1C, _P1C]):
    for _u in range(16):
        for _w in range(16):
            _S4_np[_q * 256 + _u * 16 + _w, _q * 16 + _w] = _sc


def _silu(v):
    return v * jax.nn.sigmoid(v)


def _prep_body(x_ref, w_ref, b_ref, p_ref, out_ref):
    f32 = jnp.float32
    P = p_ref[...]
    xp = jnp.dot(x_ref[...], P, preferred_element_type=f32)
    w_p = jnp.dot(w_ref[...], P, preferred_element_type=f32)
    b_p = jnp.dot(b_ref[...], P, preferred_element_type=f32)
    s = xp[:, :MUL]
    mean = jnp.mean(s, axis=1, keepdims=True)
    var = jnp.mean((s - mean) ** 2, axis=1, keepdims=True)
    sn = (s - mean) * lax.rsqrt(var + EPS)
    vx = xp[:, 16:32]
    vy = xp[:, 32:48]
    vz = xp[:, 48:64]
    ninv = lax.rsqrt((vx * vx + vy * vy + vz * vz) * (1.0 / 3.0) + EPS)
    out = jnp.concatenate([sn, vx * ninv, vy * ninv, vz * ninv], axis=1)
    out = out * w_p + b_p
    out_ref[...] = jnp.concatenate(
        [out, jnp.zeros((N, 64), f32)], axis=1)


def _prep(x, nw, nb, Pm):
    return pl.pallas_call(
        _prep_body,
        out_shape=jax.ShapeDtypeStruct((N, W128), jnp.float32),
    )(x, nw, nb, Pm)


def _gather_sc(table, idx2d):
    mesh = plsc.VectorSubcoreMesh(core_axis_name="c", subcore_axis_name="s")

    @functools.partial(
        pl.kernel,
        mesh=mesh,
        out_type=jax.ShapeDtypeStruct((E_PAD, W128), jnp.float32),
        scratch_types=[
            pltpu.VMEM((CH_W, CHUNK), jnp.int32),
            pltpu.VMEM((GROUP_E, W128), jnp.float32),
            pltpu.SemaphoreType.DMA,
        ],
    )
    def gk(tab_hbm, idx_hbm, out_hbm, idx_v, gbuf, sem):
        c = lax.axis_index("c")
        s = lax.axis_index("s")
        wid = s * NC + c
        pltpu.sync_copy(idx_hbm.at[pl.ds(wid * CH_W, CH_W), :], idx_v)
        ebase = wid * E_W
        for g in range(GROUPS):
            cps = []
            for j in range(CH_PER_GROUP):
                cps.append(pltpu.async_copy(
                    tab_hbm.at[idx_v.at[g * CH_PER_GROUP + j]],
                    gbuf.at[pl.ds(j * CHUNK, CHUNK), :],
                    sem,
                ))
            for cp in cps:
                cp.wait()
            pltpu.sync_copy(
                gbuf, out_hbm.at[pl.ds(ebase + g * GROUP_E, GROUP_E), :])

    return gk(table, idx2d)


def _main_body(rbf_ref, aux_ref, xs_ref, w1_ref, b1_ref, w2_ref,
               b2_ref, w3_ref, b3_ref, gw1_ref, gb1_ref, gw2_ref, gb2_ref,
               r4_ref, s4_ref, out_ref):
    f32 = jnp.float32
    rbf = rbf_ref[...]
    h = _silu(jnp.dot(rbf, w1_ref[...], preferred_element_type=f32)
              + b1_ref[...])
    h = _silu(jnp.dot(h, w2_ref[...], preferred_element_type=f32)
              + b2_ref[...])
    tpw = jnp.dot(h, w3_ref[...], preferred_element_type=f32) + b3_ref[...]

    xs = xs_ref[...]
    aux = aux_ref[...]
    sh0 = aux[:, 0:1]
    shx = aux[:, 1:2]
    shy = aux[:, 2:3]
    shz = aux[:, 3:4]
    elen = aux[:, 4:5]
    t = elen - 0.5
    t2 = t * t
    sp = t * (_SINP[0] + t2 * (_SINP[1] + t2 * (_SINP[2] + t2 * (
        _SINP[3] + t2 * _SINP[4]))))
    cut = (0.5 - 0.5 * sp) * (elen <= CUTOFF).astype(f32)

    XR = jnp.dot(xs[:, :64], r4_ref[...], preferred_element_type=f32)
    sr = XR[:, 0:256]
    vxr = XR[:, 256:512]
    vyr = XR[:, 512:768]
    vzr = XR[:, 768:1024]

    ar = sr * sh0
    br = vxr * shx + vyr * shy + vzr * shz
    t0 = tpw[:, 0:256] * ar + _C3 * (tpw[:, 768:1024] * br)
    c011 = tpw[:, 256:512] * sr
    tq = tpw[:, 512:768]
    txp = c011 * shx + (tq * vxr) * sh0
    typ = c011 * shy + (tq * vyr) * sh0
    tzp = c011 * shz + (tq * vzr) * sh0
    T = jnp.concatenate([t0, txp, typ, tzp], axis=1)
    m = jnp.dot(T, s4_ref[...], preferred_element_type=f32)

    g = _silu(jnp.dot(rbf, gw1_ref[...], preferred_element_type=f32)
              + gb1_ref[...])
    gw = jax.nn.sigmoid(jnp.dot(g, gw2_ref[...], preferred_element_type=f32)
                        + gb2_ref[...])
    ew = cut * gw

    m = m * ew
    ew16 = jnp.broadcast_to(ew, (m.shape[0], 16))
    pad48 = jnp.zeros((m.shape[0], 48), f32)
    out_ref[...] = jnp.concatenate([m, ew16, pad48], axis=1)


def _main(rbf, aux, xs, w1, b1, w2, b2, w3, b3, gw1, gb1, gw2, gb2,
          R4m, S4m):
    blk = lambda shp: pl.BlockSpec(shp, lambda i: (0, 0))
    ebk = lambda w: pl.BlockSpec((EB, w), lambda i: (i, 0))
    return pl.pallas_call(
        _main_body,
        grid=(N_EB,),
        in_specs=[
            ebk(16), ebk(8), ebk(W128),
            blk((16, 64)), blk((1, 64)), blk((64, 64)), blk((1, 64)),
            blk((64, 1024)), blk((1, 1024)),
            blk((16, 64)), blk((1, 64)), blk((64, 1)), blk((1, 1)),
            blk((64, 1024)), blk((1024, 64)),
        ],
        out_specs=pl.BlockSpec((EB, W128), lambda i: (i, 0)),
        out_shape=jax.ShapeDtypeStruct((E_PAD, W128), jnp.float32),
    )(rbf, aux, xs, w1, b1, w2, b2, w3, b3, gw1, gb1, gw2, gb2,
      R4m, S4m)


def _scatter_sc(m_ext, dst2d, zstripe):
    mesh = plsc.VectorSubcoreMesh(core_axis_name="c", subcore_axis_name="s")

    @functools.partial(
        pl.kernel,
        mesh=mesh,
        out_type=jax.ShapeDtypeStruct((NC, N_PAD, W128), jnp.float32),
        scratch_types=[
            pltpu.VMEM((CH_W, CHUNK), jnp.int32),
            pltpu.VMEM((CHUNK, W128), jnp.float32),
            pltpu.VMEM((CHUNK, W128), jnp.float32),
            pltpu.VMEM_SHARED((N_PAD, W128), jnp.float32),
            pltpu.SemaphoreType.DMA,
            pltpu.SemaphoreType.DMA,
            pltpu.SemaphoreType.DMA,
            pltpu.SemaphoreType.DMA,
        ],
    )
    def sk(m_hbm, idx_hbm, z_hbm, out_hbm, idx_v, mb0, mb1, acc,
           ls0, ls1, as0, as1):
        c = lax.axis_index("c")
        s = lax.axis_index("s")
        wid = s * NC + c
        pltpu.sync_copy(idx_hbm.at[pl.ds(wid * CH_W, CH_W), :], idx_v)
        pltpu.sync_copy(z_hbm, acc.at[pl.ds(s * STRIPE, STRIPE), :])
        plsc.subcore_barrier()
        ebase = wid * E_W
        bufs = (mb0, mb1)
        lsems = (ls0, ls1)
        asems = (as0, as1)
        lds = [None, None]
        ads = [None, None]

        def load(t):
            q = t % 2
            lds[q] = pltpu.async_copy(
                m_hbm.at[pl.ds(ebase + t * CHUNK, CHUNK), :], bufs[q],
                lsems[q])

        load(0)
        for j in range(CH_W):
            p = j % 2
            nj = j + 1
            if nj < CH_W:
                q = nj % 2
                if ads[q] is not None:
                    ads[q].wait()
                    ads[q] = None
                load(nj)
            lds[p].wait()
            ads[p] = pltpu.async_copy(
                bufs[p], acc.at[idx_v.at[j]], asems[p], add=True)
        for q in range(2):
            if ads[q] is not None:
                ads[q].wait()
        plsc.subcore_barrier()
        pltpu.sync_copy(
            acc.at[pl.ds(s * STRIPE, STRIPE), :],
            out_hbm.at[c, pl.ds(s * STRIPE, STRIPE), :],
        )

    return sk(m_ext, dst2d, zstripe)


def _epi_body(aggc_ref, x_ref, xn_ref, mws_ref, mwg_ref, mwv_ref,
              uw0_ref, uw1_ref, sw0_ref, sw1_ref, rs_ref, pt_ref, out_ref):
    f32 = jnp.float32
    agg = aggc_ref[0] + aggc_ref[1]
    agg = agg[:N, :]
    den = jnp.maximum(agg[:, 64:65], 1e-8)
    a = agg[:, :64] / den
    a_s = a[:, :16]
    a_vx = a[:, 16:32]
    a_vy = a[:, 32:48]
    a_vz = a[:, 48:64]

    scal = _silu(jnp.dot(a_s, mws_ref[...], preferred_element_type=f32)
                 * _SCL)
    gts = jax.nn.sigmoid(jnp.dot(a_s, mwg_ref[...],
                                 preferred_element_type=f32) * _SCL)
    mwv = mwv_ref[...]
    vex = gts * (jnp.dot(a_vx, mwv, preferred_element_type=f32) * _SCL)
    vey = gts * (jnp.dot(a_vy, mwv, preferred_element_type=f32) * _SCL)
    vez = gts * (jnp.dot(a_vz, mwv, preferred_element_type=f32) * _SCL)

    xn = xn_ref[...]
    sw0 = sw0_ref[...]
    sw1 = sw1_ref[...]
    uw0 = uw0_ref[...]
    uw1 = uw1_ref[...]
    o_s = (jnp.dot(xn[:, :16], sw0, preferred_element_type=f32)
           + jnp.dot(scal, uw0, preferred_element_type=f32)) * _SCL
    o_vx = (jnp.dot(xn[:, 16:32], sw1, preferred_element_type=f32)
            + jnp.dot(vex, uw1, preferred_element_type=f32)) * _SCL
    o_vy = (jnp.dot(xn[:, 32:48], sw1, preferred_element_type=f32)
            + jnp.dot(vey, uw1, preferred_element_type=f32)) * _SCL
    o_vz = (jnp.dot(xn[:, 48:64], sw1, preferred_element_type=f32)
            + jnp.dot(vez, uw1, preferred_element_type=f32)) * _SCL
    out_p = jnp.concatenate([o_s, o_vx, o_vy, o_vz], axis=1)
    out = jnp.dot(out_p, pt_ref[...], preferred_element_type=f32)
    out_ref[...] = x_ref[...] + rs_ref[0, 0] * out


def _epilogue(aggc, x, xn_p, mws, mwg, mwv, uw0, uw1, sw0, sw1, rs, PTm):
    return pl.pallas_call(
        _epi_body,
        out_shape=jax.ShapeDtypeStruct((N, D), jnp.float32),
    )(aggc, x, xn_p, mws, mwg, mwv, uw0, uw1, sw0, sw1, rs, PTm)


def kernel(x, edge_src, edge_dst, edge_sh, edge_rbf, edge_len, norm_w,
           norm_b, mlp_w1, mlp_b1, mlp_w2, mlp_b2, mlp_w3, mlp_b3, gate_w1,
           gate_b1, gate_w2, gate_b2, msg_ws, msg_wg, msg_wv, upd_w0, upd_w1,
           self_w0, self_w1, res_scale):
    f32 = jnp.float32
    pad = E_PAD - E
    src_p = jnp.pad(edge_src.astype(jnp.int32), (0, pad)).reshape(TOT_CH,
                                                                  CHUNK)
    dst_p = jnp.pad(edge_dst.astype(jnp.int32), (0, pad),
                    constant_values=TRASH).reshape(TOT_CH, CHUNK)

    zstripe = jnp.zeros((STRIPE, W128), f32)
    Pm = jnp.asarray(_P_np)
    PTm = jnp.asarray(_P_np.T)
    R4m = jnp.asarray(_R4_np)
    S4m = jnp.asarray(_S4_np)

    xn_p = _prep(x, norm_w.reshape(1, D), norm_b.reshape(1, D), Pm)
    aux = jnp.concatenate(
        [edge_sh, edge_len[:, None], jnp.zeros((E, 3), f32)], axis=1)
    xs = _gather_sc(xn_p, src_p)
    m_ext = _main(edge_rbf, aux, xs,
                  mlp_w1, mlp_b1.reshape(1, -1), mlp_w2,
                  mlp_b2.reshape(1, -1), mlp_w3, mlp_b3.reshape(1, -1),
                  gate_w1, gate_b1.reshape(1, -1), gate_w2,
                  gate_b2.reshape(1, -1), R4m, S4m)
    aggc = _scatter_sc(m_ext, dst_p, zstripe)
    return _epilogue(aggc, x, xn_p, msg_ws, msg_wg, msg_wv, upd_w0,
                     upd_w1, self_w0, self_w1,
                     res_scale.reshape(1, 1), PTm)
